# baseline (speedup 1.0000x reference)
import functools

import jax
import jax.numpy as jnp
from jax import lax
from jax.experimental import pallas as pl
from jax.experimental.pallas import tpu as pltpu
from jax.experimental.pallas import tpu_sc as plsc

NC = 2
NS = 16
NW = NC * NS
LANES = 16
CAP = 8192
PG = 2
P = PG * LANES
UNROLL = 4
BS = 256


def _sc_spmm(n_pixels, c_out, c_in):
    n_groups = n_pixels // LANES
    chunks_per_worker = n_groups // (PG * NW)
    assert chunks_per_worker * PG * NW == n_groups
    assert CAP % BS == 0 and BS % UNROLL == 0

    mesh = plsc.VectorSubcoreMesh(core_axis_name="c", subcore_axis_name="s")

    @functools.partial(
        pl.kernel,
        out_type=jax.ShapeDtypeStruct((c_out, n_groups, LANES), jnp.float32),
        mesh=mesh,
        scratch_types=[
            pltpu.VMEM_SHARED((CAP,), jnp.int32),
            pltpu.VMEM_SHARED((CAP,), jnp.int32),
            pltpu.VMEM_SHARED((CAP,), jnp.float32),
            pltpu.VMEM((c_in, PG, LANES), jnp.float32),
            pltpu.VMEM((c_out, PG, LANES), jnp.float32),
            pltpu.SMEM((BS,), jnp.int32),
            pltpu.SMEM((BS,), jnp.int32),
            pltpu.SMEM((BS,), jnp.float32),
            pltpu.SMEM((1,), jnp.int32),
            pltpu.VMEM_SHARED((1,), jnp.int32),
        ],
        compiler_params=pltpu.CompilerParams(use_tc_tiling_on_sc=False),
    )
    def spmm(x_hbm, o_hbm, i_hbm, w_hbm, nb_hbm, out_hbm,
             o_sp, i_sp, w_sp, x_vm, y_vm, o_sm, i_sm, w_sm, nb_sm, nb_sp):
        sid = lax.axis_index("s")
        wid = sid * NC + lax.axis_index("c")

        @pl.when(sid == 0)
        def _():
            pltpu.sync_copy(o_hbm, o_sp)
            pltpu.sync_copy(i_hbm, i_sp)
            pltpu.sync_copy(w_hbm, w_sp)
            pltpu.sync_copy(nb_hbm, nb_sp)

        plsc.subcore_barrier()
        pltpu.sync_copy(nb_sp, nb_sm)
        nbatches = nb_sm[0]

        zeros = jnp.zeros((LANES,), jnp.float32)

        def chunk_body(t, _):
            gbase = (wid * chunks_per_worker + t) * PG
            pltpu.sync_copy(x_hbm.at[:, pl.ds(gbase, PG), :], x_vm)

            def zero_body(r, _):
                for c in range(PG):
                    y_vm[r, c] = zeros
                return 0

            lax.fori_loop(0, c_out, zero_body, 0)

            def batch_body(b, _):
                pltpu.sync_copy(o_sp.at[pl.ds(b * BS, BS)], o_sm)
                pltpu.sync_copy(i_sp.at[pl.ds(b * BS, BS)], i_sm)
                pltpu.sync_copy(w_sp.at[pl.ds(b * BS, BS)], w_sm)

                def nz_body(k, _):
                    for u in range(UNROLL):
                        kk = k * UNROLL + u
                        o = o_sm[kk]
                        i = i_sm[kk]
                        w = w_sm[kk]
                        for c in range(PG):
                            xv = x_vm[i, c]
                            plsc.addupdate(y_vm.at[o, c], xv * w)
                    return 0

                lax.fori_loop(0, BS // UNROLL, nz_body, 0)
                return 0

            lax.fori_loop(0, nbatches, batch_body, 0)
            pltpu.sync_copy(y_vm, out_hbm.at[:, pl.ds(gbase, PG), :])
            return 0

        lax.fori_loop(0, chunks_per_worker, chunk_body, 0)

    return spmm


def kernel(x, W):
    c_in = x.shape[1]
    h, w_dim = x.shape[2], x.shape[3]
    n = h * w_dim
    c_out = W.shape[0]
    x_flat = x.reshape(c_in, n // LANES, LANES)

    flat = W.reshape(-1)
    mask = flat != 0.0
    pos = jnp.cumsum(mask.astype(jnp.int32)) - 1
    nnz = pos[-1] + 1
    dest = jnp.where(mask, pos, CAP)
    idx = jnp.zeros((CAP,), jnp.int32).at[dest].set(
        jnp.arange(flat.shape[0], dtype=jnp.int32), mode="drop")
    valid = jnp.arange(CAP, dtype=jnp.int32) < nnz
    w_vals = jnp.where(valid, flat[idx], 0.0)
    o_idx = jnp.where(valid, idx // c_in, 0)
    i_idx = jnp.where(valid, idx % c_in, 0)
    nbatches = ((nnz + BS - 1) // BS).reshape(1).astype(jnp.int32)

    y = _sc_spmm(n, c_out, c_in)(x_flat, o_idx, i_idx, w_vals, nbatches)
    return y.reshape(1, c_out, h, w_dim)


# speedup vs baseline: 3.7230x; 3.7230x over previous
import functools

import jax
import jax.numpy as jnp
from jax import lax
from jax.experimental import pallas as pl
from jax.experimental.pallas import tpu as pltpu
from jax.experimental.pallas import tpu_sc as plsc

NC = 2
NS = 16
NW = NC * NS
LANES = 16
P = 128
SEG = 128
NSEG = 6
REG = 1280
CAPT = NSEG * REG
BS = 256
UNROLL = 4


def _sc_spmm(n_pixels, c_out, c_in):
    n_chunks = n_pixels // P
    assert n_chunks * P == n_pixels
    base_chunks = n_chunks // NW
    extra = n_chunks - base_chunks * NW

    mesh = plsc.VectorSubcoreMesh(core_axis_name="c", subcore_axis_name="s")

    @functools.partial(
        pl.kernel,
        out_type=jax.ShapeDtypeStruct((c_out, n_pixels), jnp.float32),
        mesh=mesh,
        scratch_types=[
            pltpu.VMEM((c_out, P), jnp.float32),
            pltpu.VMEM((SEG, P), jnp.float32),
            pltpu.VMEM((CAPT,), jnp.int32),
            pltpu.VMEM((CAPT,), jnp.float32),
            pltpu.VMEM_SHARED((CAPT,), jnp.int32),
            pltpu.VMEM_SHARED((CAPT,), jnp.float32),
            pltpu.SMEM((BS,), jnp.int32),
            pltpu.SMEM((BS,), jnp.float32),
        ],
        compiler_params=pltpu.CompilerParams(use_tc_tiling_on_sc=True,
                                             needs_layout_passes=False),
    )
    def spmm(x_hbm, w_hbm, out_hbm,
             y_vm, x_vm, oi_vm, wv_vm, oi_sp, wv_sp, oi_sm, wv_sm):
        sid = lax.axis_index("s")
        wid = sid * NC + lax.axis_index("c")

        zeros = jnp.zeros((LANES,), jnp.float32)
        izeros = jnp.zeros((LANES,), jnp.int32)
        lanes = lax.iota(jnp.int32, LANES)

        @pl.when(sid == 0)
        def _():
            @plsc.parallel_loop(0, CAPT // LANES, unroll=8)
            def _(g):
                oi_vm[pl.ds(g * LANES, LANES)] = izeros
                wv_vm[pl.ds(g * LANES, LANES)] = zeros

            for cb in range(NSEG):
                pltpu.sync_copy(w_hbm.at[:, pl.ds(cb * SEG, SEG)], y_vm)

                def row_body(r, wp):
                    for j in range(SEG // LANES):
                        v = y_vm[r, pl.ds(j * LANES, LANES)]
                        m = v != 0.0
                        pos = plsc.cumsum(
                            jnp.where(m, jnp.int32(1), jnp.int32(0)))
                        idx = wp + pos - 1 + cb * REG
                        oi = (r * P + j * LANES) + lanes
                        plsc.store_scatter(oi_vm, [idx], oi, mask=m)
                        plsc.store_scatter(wv_vm, [idx], v, mask=m)
                        wp = wp + plsc.all_reduce_population_count(m)
                    return wp

                lax.fori_loop(0, c_out, row_body, izeros)

            pltpu.sync_copy(oi_vm, oi_sp)
            pltpu.sync_copy(wv_vm, wv_sp)

        plsc.subcore_barrier()

        my_count = base_chunks + jnp.where(wid < extra, 1, 0)
        my_start = wid * base_chunks + jnp.minimum(wid, extra)

        def chunk_body(t, _):
            base = (my_start + t) * P

            @plsc.parallel_loop(0, c_out, unroll=8)
            def _(r):
                for c in range(P // LANES):
                    y_vm[r, pl.ds(c * LANES, LANES)] = zeros

            for seg in range(NSEG):
                pltpu.sync_copy(
                    x_hbm.at[pl.ds(seg * SEG, SEG), pl.ds(base, P)], x_vm)
                for b in range(REG // BS):
                    off = seg * REG + b * BS
                    pltpu.sync_copy(oi_sp.at[pl.ds(off, BS)], oi_sm)
                    pltpu.sync_copy(wv_sp.at[pl.ds(off, BS)], wv_sm)

                    @plsc.parallel_loop(0, BS, unroll=UNROLL)
                    def _(kk):
                        oi = oi_sm[kk]
                        w = wv_sm[kk]
                        o = lax.shift_right_logical(oi, 7)
                        i_rel = lax.bitwise_and(oi, P - 1)
                        for c in range(P // LANES):
                            xv = x_vm[i_rel, pl.ds(c * LANES, LANES)]
                            plsc.addupdate(
                                y_vm.at[o, pl.ds(c * LANES, LANES)], xv * w)

            pltpu.sync_copy(y_vm, out_hbm.at[:, pl.ds(base, P)])
            return 0

        lax.fori_loop(0, my_count, chunk_body, 0)

    return spmm


def kernel(x, W):
    c_in = x.shape[1]
    h, w_dim = x.shape[2], x.shape[3]
    n = h * w_dim
    c_out = W.shape[0]
    x_flat = x.reshape(c_in, n)
    y = _sc_spmm(n, c_out, c_in)(x_flat, W)
    return y.reshape(1, c_out, h, w_dim)


# speedup vs baseline: 4.6690x; 1.2541x over previous
import functools

import jax
import jax.numpy as jnp
from jax import lax
from jax.experimental import pallas as pl
from jax.experimental.pallas import tpu as pltpu
from jax.experimental.pallas import tpu_sc as plsc

NC = 2
NS = 16
NW = NC * NS
LANES = 16
P = 128
SEG = 128
NSEG = 6
REG = 1280
CAPT = NSEG * REG
BS = 256
UNROLL = 4


def _sc_spmm(n_pixels, c_out, c_in):
    n_chunks = n_pixels // P
    assert n_chunks * P == n_pixels
    base_chunks = n_chunks // NW
    extra = n_chunks - base_chunks * NW

    mesh = plsc.VectorSubcoreMesh(core_axis_name="c", subcore_axis_name="s")

    @functools.partial(
        pl.kernel,
        out_type=jax.ShapeDtypeStruct((c_out, n_pixels), jnp.float32),
        mesh=mesh,
        scratch_types=[
            pltpu.VMEM((c_out, P), jnp.float32),
            pltpu.VMEM((SEG, P), jnp.float32),
            pltpu.VMEM((CAPT,), jnp.int32),
            pltpu.VMEM((CAPT,), jnp.float32),
            pltpu.VMEM_SHARED((CAPT,), jnp.int32),
            pltpu.VMEM_SHARED((CAPT,), jnp.float32),
            pltpu.SMEM((BS,), jnp.int32),
            pltpu.SMEM((BS,), jnp.float32),
        ],
        compiler_params=pltpu.CompilerParams(use_tc_tiling_on_sc=True,
                                             needs_layout_passes=False),
    )
    def spmm(x_hbm, w_hbm, out_hbm,
             y_vm, x_vm, oi_vm, wv_vm, oi_sp, wv_sp, oi_sm, wv_sm):
        sid = lax.axis_index("s")
        wid = sid * NC + lax.axis_index("c")

        zeros = jnp.zeros((LANES,), jnp.float32)
        izeros = jnp.zeros((LANES,), jnp.int32)
        lanes = lax.iota(jnp.int32, LANES)

        @pl.when(sid < NSEG)
        def _():
            cb = sid

            @plsc.parallel_loop(0, REG // LANES, unroll=8)
            def _(g):
                oi_vm[pl.ds(cb * REG + g * LANES, LANES)] = izeros
                wv_vm[pl.ds(cb * REG + g * LANES, LANES)] = zeros

            pltpu.sync_copy(
                w_hbm.at[:, pl.ds(cb * SEG, SEG)], y_vm)

            def row_body(r, wp):
                for j in range(SEG // LANES):
                    v = y_vm[r, pl.ds(j * LANES, LANES)]
                    m = v != 0.0
                    pos = plsc.cumsum(
                        jnp.where(m, jnp.int32(1), jnp.int32(0)))
                    idx = wp + pos - 1 + cb * REG
                    oi = (r * P + j * LANES) + lanes
                    plsc.store_scatter(oi_vm, [idx], oi, mask=m)
                    plsc.store_scatter(wv_vm, [idx], v, mask=m)
                    wp = wp + plsc.all_reduce_population_count(m)
                return wp

            lax.fori_loop(0, c_out, row_body, izeros)

            pltpu.sync_copy(oi_vm.at[pl.ds(cb * REG, REG)],
                            oi_sp.at[pl.ds(cb * REG, REG)])
            pltpu.sync_copy(wv_vm.at[pl.ds(cb * REG, REG)],
                            wv_sp.at[pl.ds(cb * REG, REG)])

        plsc.subcore_barrier()

        my_count = base_chunks + jnp.where(wid < extra, 1, 0)
        my_start = wid * base_chunks + jnp.minimum(wid, extra)

        def chunk_body(t, _):
            base = (my_start + t) * P

            @plsc.parallel_loop(0, c_out, unroll=8)
            def _(r):
                for c in range(P // LANES):
                    y_vm[r, pl.ds(c * LANES, LANES)] = zeros

            for seg in range(NSEG):
                pltpu.sync_copy(
                    x_hbm.at[pl.ds(seg * SEG, SEG), pl.ds(base, P)], x_vm)
                for b in range(REG // BS):
                    off = seg * REG + b * BS
                    pltpu.sync_copy(oi_sp.at[pl.ds(off, BS)], oi_sm)
                    pltpu.sync_copy(wv_sp.at[pl.ds(off, BS)], wv_sm)

                    @plsc.parallel_loop(0, BS, unroll=UNROLL)
                    def _(kk):
                        oi = oi_sm[kk]
                        w = wv_sm[kk]
                        o = lax.shift_right_logical(oi, 7)
                        i_rel = lax.bitwise_and(oi, P - 1)
                        for c in range(P // LANES):
                            xv = x_vm[i_rel, pl.ds(c * LANES, LANES)]
                            plsc.addupdate(
                                y_vm.at[o, pl.ds(c * LANES, LANES)], xv * w)

            pltpu.sync_copy(y_vm, out_hbm.at[:, pl.ds(base, P)])
            return 0

        lax.fori_loop(0, my_count, chunk_body, 0)

    return spmm


def kernel(x, W):
    c_in = x.shape[1]
    h, w_dim = x.shape[2], x.shape[3]
    n = h * w_dim
    c_out = W.shape[0]
    x_flat = x.reshape(c_in, n)
    y = _sc_spmm(n, c_out, c_in)(x_flat, W)
    return y.reshape(1, c_out, h, w_dim)
